# baseline (speedup 1.0000x reference)
import functools

import jax
import jax.numpy as jnp
from jax import lax
from jax.experimental import pallas as pl
from jax.experimental.pallas import tpu as pltpu
from jax.experimental.pallas import tpu_sc as plsc

N_NODES = 10000
N_PAD = 10240
N_EDGES = 320000
N_GRAPHS = 64
DW = 128
DEGW = 16

NC = 2
NS = 16
NW = NC * NS
CHUNK = 128
ROWS_PER_TILE = 79
E_PAD = NW * ROWS_PER_TILE * CHUNK
NODES_PER_TILE = N_PAD // NS

_HP = lax.Precision.HIGHEST


def _sc_mesh():
    return plsc.VectorSubcoreMesh(core_axis_name="c", subcore_axis_name="s")


@functools.partial(
    pl.kernel,
    mesh=_sc_mesh(),
    out_type=jax.ShapeDtypeStruct((NC, N_PAD, DEGW), jnp.float32),
    scratch_types=[
        pltpu.VMEM((ROWS_PER_TILE, CHUNK), jnp.int32),
        pltpu.VMEM((CHUNK, DEGW), jnp.float32),
        pltpu.VMEM_SHARED((N_PAD, DEGW), jnp.float32),
        pltpu.SemaphoreType.DMA,
    ],
    compiler_params=pltpu.CompilerParams(use_tc_tiling_on_sc=False),
)
def _deg_kernel(dst_hbm, zeros_hbm, ones_hbm, out_hbm, dst_v, ones_v, acc_sh, sem):
    cid = lax.axis_index("c")
    sid = lax.axis_index("s")
    wid = cid * NS + sid
    nslice = pl.ds(sid * NODES_PER_TILE, NODES_PER_TILE)
    pltpu.sync_copy(zeros_hbm.at[nslice], acc_sh.at[nslice])
    pltpu.sync_copy(dst_hbm.at[wid], dst_v)
    pltpu.sync_copy(ones_hbm, ones_v)
    plsc.subcore_barrier()

    WIN = 8

    def body(c, carry):
        pltpu.make_async_copy(
            ones_v, acc_sh.at[dst_v.at[c]], sem).start(add=True)

        @pl.when(c >= WIN)
        def _():
            pltpu.make_async_copy(ones_v, acc_sh.at[dst_v.at[0]], sem).wait()
        return carry

    lax.fori_loop(0, ROWS_PER_TILE, body, 0)

    def drain(c, carry):
        pltpu.make_async_copy(ones_v, acc_sh.at[dst_v.at[0]], sem).wait()
        return carry

    lax.fori_loop(0, WIN, drain, 0)
    plsc.subcore_barrier()
    pltpu.sync_copy(acc_sh.at[nslice], out_hbm.at[cid, nslice])


def _make_edge_kernel(D):
    @functools.partial(
        pl.kernel,
        mesh=_sc_mesh(),
        out_type=jax.ShapeDtypeStruct((NC, N_PAD, D), jnp.float32),
        scratch_types=[
            pltpu.VMEM((ROWS_PER_TILE, CHUNK), jnp.int32),
            pltpu.VMEM((ROWS_PER_TILE, CHUNK), jnp.int32),
            pltpu.VMEM((2, CHUNK, D), jnp.float32),
            pltpu.VMEM_SHARED((N_PAD, D), jnp.float32),
            pltpu.SemaphoreType.DMA,
            pltpu.SemaphoreType.DMA,
        ],
        compiler_params=pltpu.CompilerParams(use_tc_tiling_on_sc=False),
    )
    def _edge_kernel(src_hbm, dst_hbm, g_hbm, zeros_hbm, out_hbm,
                     src_v, dst_v, rows_v, acc_sh, sem0, sem1):
        cid = lax.axis_index("c")
        sid = lax.axis_index("s")
        wid = cid * NS + sid
        nslice = pl.ds(sid * NODES_PER_TILE, NODES_PER_TILE)
        pltpu.sync_copy(zeros_hbm.at[nslice], acc_sh.at[nslice])
        pltpu.sync_copy(src_hbm.at[wid], src_v)
        pltpu.sync_copy(dst_hbm.at[wid], dst_v)
        plsc.subcore_barrier()

        pltpu.make_async_copy(g_hbm.at[src_v.at[0]], rows_v.at[0], sem0).start()

        def body(i, carry):
            c0 = i * 2
            pltpu.make_async_copy(
                g_hbm.at[src_v.at[c0 + 1]], rows_v.at[1], sem1).start()
            pltpu.make_async_copy(
                g_hbm.at[src_v.at[c0]], rows_v.at[0], sem0).wait()
            pltpu.sync_copy(rows_v.at[0], acc_sh.at[dst_v.at[c0]], add=True)
            pltpu.make_async_copy(
                g_hbm.at[src_v.at[c0 + 2]], rows_v.at[0], sem0).start()
            pltpu.make_async_copy(
                g_hbm.at[src_v.at[c0 + 1]], rows_v.at[1], sem1).wait()
            pltpu.sync_copy(rows_v.at[1], acc_sh.at[dst_v.at[c0 + 1]], add=True)
            return carry

        lax.fori_loop(0, (ROWS_PER_TILE - 1) // 2, body, 0)
        last = ROWS_PER_TILE - 1
        pltpu.make_async_copy(g_hbm.at[src_v.at[last]], rows_v.at[0], sem0).wait()
        pltpu.sync_copy(rows_v.at[0], acc_sh.at[dst_v.at[last]], add=True)

        plsc.subcore_barrier()
        pltpu.sync_copy(acc_sh.at[nslice], out_hbm.at[cid, nslice])

    return _edge_kernel


_edge96 = _make_edge_kernel(96)
_edge64 = _make_edge_kernel(64)
_edge32 = _make_edge_kernel(32)


def _dinv_from(deg_ref):
    deg = deg_ref[0, :, 0:1] + deg_ref[1, :, 0:1] + 1.0
    return lax.rsqrt(deg)


def _m_body(x_ref, w_ref, h_ref):
    h_ref[...] = jnp.dot(x_ref[...], w_ref[...], precision=_HP,
                         preferred_element_type=jnp.float32)


_tc_m = pl.pallas_call(
    _m_body, out_shape=jax.ShapeDtypeStruct((N_PAD, 96), jnp.float32))


def _s_body(h_ref, deg_ref, g_ref):
    g_ref[...] = h_ref[...] * _dinv_from(deg_ref)


_tc_s = pl.pallas_call(
    _s_body, out_shape=jax.ShapeDtypeStruct((N_PAD, 96), jnp.float32))


def _make_tc_d(Dt, Dn):
    def body(acc_ref, g_ref, deg_ref, b_ref, w_ref, out_ref):
        dinv = _dinv_from(deg_ref)
        t = (acc_ref[0] + acc_ref[1] + g_ref[...]) * dinv
        t = jnp.maximum(t[:, :Dt] + b_ref[...], 0.0)
        out_ref[...] = jnp.dot(t, w_ref[...], precision=_HP,
                               preferred_element_type=jnp.float32) * dinv

    return pl.pallas_call(
        body, out_shape=jax.ShapeDtypeStruct((N_PAD, Dn), jnp.float32))


_tc_d1 = _make_tc_d(84, 64)
_tc_d2 = _make_tc_d(64, 32)


def _d3_body(acc_ref, g_ref, deg_ref, b_ref, batch_ref, wl_ref, bl_ref, out_ref):
    dinv = _dinv_from(deg_ref)
    h = (acc_ref[0] + acc_ref[1] + g_ref[...]) * dinv
    h = jnp.maximum(h + b_ref[...], 0.0)
    gid = batch_ref[...]
    oh = (gid == lax.broadcasted_iota(jnp.int32, (1, N_GRAPHS), 1))
    oh = oh.astype(jnp.float32)
    sums = lax.dot_general(oh, h, (((0,), (0,)), ((), ())),
                           precision=_HP, preferred_element_type=jnp.float32)
    cnt = jnp.sum(oh, axis=0).reshape(N_GRAPHS, 1)
    pooled = sums / jnp.maximum(cnt, 1.0)
    out_ref[...] = jnp.dot(pooled, wl_ref[...], precision=_HP,
                           preferred_element_type=jnp.float32) + bl_ref[...]


_tc_d3 = pl.pallas_call(
    _d3_body, out_shape=jax.ShapeDtypeStruct((N_GRAPHS, 10), jnp.float32))


@jax.jit
def kernel(x, edge_index, batch, W1, b1, W2, b2, W3, b3, Wl, bl):
    ei = edge_index.astype(jnp.int32)
    ei = jnp.pad(ei, ((0, 0), (0, E_PAD - N_EDGES)),
                 constant_values=N_PAD - 1)
    src = ei[0].reshape(NW, ROWS_PER_TILE, CHUNK)
    dst = ei[1].reshape(NW, ROWS_PER_TILE, CHUNK)
    batch2 = jnp.pad(batch.astype(jnp.int32), (0, N_PAD - N_NODES),
                     constant_values=-1).reshape(N_PAD, 1)
    xp = jnp.pad(x, ((0, N_PAD - N_NODES), (0, 0)))

    zdeg = jnp.zeros((N_PAD, DEGW), jnp.float32)
    ones = jnp.ones((CHUNK, DEGW), jnp.float32)
    W1p = jnp.pad(W1, ((0, 0), (0, 96 - 84)))
    h1 = _tc_m(xp, W1p)
    deg2 = _deg_kernel(dst, zdeg, ones)
    g1 = _tc_s(h1, deg2)
    acc1 = _edge96(src, dst, g1, jnp.zeros((N_PAD, 96), jnp.float32))
    g2 = _tc_d1(acc1, g1, deg2, b1.reshape(1, 84), W2)
    acc2 = _edge64(src, dst, g2, jnp.zeros((N_PAD, 64), jnp.float32))
    g3 = _tc_d2(acc2, g2, deg2, b2.reshape(1, 64), W3)
    acc3 = _edge32(src, dst, g3, jnp.zeros((N_PAD, 32), jnp.float32))
    return _tc_d3(acc3, g3, deg2, b3.reshape(1, 32), batch2,
                  Wl, bl.reshape(1, 10))


# speedup vs baseline: 1.7167x; 1.7167x over previous
import functools

import jax
import jax.numpy as jnp
from jax import lax
from jax.experimental import pallas as pl
from jax.experimental.pallas import tpu as pltpu
from jax.experimental.pallas import tpu_sc as plsc

N_NODES = 10000
N_PAD = 10240
N_EDGES = 320000
N_GRAPHS = 64
DW = 128
DEGW = 16

NC = 2
NS = 16
NW = NC * NS
CHUNK = 128
ROWS_PER_TILE = 79
E_PAD = NW * ROWS_PER_TILE * CHUNK
NODES_PER_TILE = N_PAD // NS

_HP = lax.Precision.HIGHEST


def _sc_mesh():
    return plsc.VectorSubcoreMesh(core_axis_name="c", subcore_axis_name="s")


@functools.partial(
    pl.kernel,
    mesh=_sc_mesh(),
    out_type=jax.ShapeDtypeStruct((NC, N_PAD, DEGW), jnp.float32),
    scratch_types=[
        pltpu.VMEM((ROWS_PER_TILE, CHUNK), jnp.int32),
        pltpu.VMEM((CHUNK, DEGW), jnp.float32),
        pltpu.VMEM_SHARED((N_PAD, DEGW), jnp.float32),
        pltpu.SemaphoreType.DMA,
    ],
    compiler_params=pltpu.CompilerParams(use_tc_tiling_on_sc=False),
)
def _deg_kernel(dst_hbm, zeros_hbm, ones_hbm, out_hbm, dst_v, ones_v, acc_sh, sem):
    cid = lax.axis_index("c")
    sid = lax.axis_index("s")
    wid = cid * NS + sid
    nslice = pl.ds(sid * NODES_PER_TILE, NODES_PER_TILE)
    pltpu.sync_copy(zeros_hbm.at[nslice], acc_sh.at[nslice])
    pltpu.sync_copy(dst_hbm.at[wid], dst_v)
    pltpu.sync_copy(ones_hbm, ones_v)
    plsc.subcore_barrier()

    WIN = 8

    def body(c, carry):
        pltpu.make_async_copy(
            ones_v, acc_sh.at[dst_v.at[c]], sem).start(add=True)

        @pl.when(c >= WIN)
        def _():
            pltpu.make_async_copy(ones_v, acc_sh.at[dst_v.at[0]], sem).wait()
        return carry

    lax.fori_loop(0, ROWS_PER_TILE, body, 0)

    def drain(c, carry):
        pltpu.make_async_copy(ones_v, acc_sh.at[dst_v.at[0]], sem).wait()
        return carry

    lax.fori_loop(0, WIN, drain, 0)
    plsc.subcore_barrier()
    pltpu.sync_copy(acc_sh.at[nslice], out_hbm.at[cid, nslice])


def _make_edge_kernel(D):
    @functools.partial(
        pl.kernel,
        mesh=_sc_mesh(),
        out_type=jax.ShapeDtypeStruct((NC, N_PAD, D), jnp.float32),
        scratch_types=[
            pltpu.VMEM((ROWS_PER_TILE, CHUNK), jnp.int32),
            pltpu.VMEM((ROWS_PER_TILE, CHUNK), jnp.int32),
            pltpu.VMEM((2, CHUNK, D), jnp.float32),
            pltpu.VMEM_SHARED((N_PAD, D), jnp.float32),
            pltpu.SemaphoreType.DMA,
            pltpu.SemaphoreType.DMA,
        ],
        compiler_params=pltpu.CompilerParams(use_tc_tiling_on_sc=False),
    )
    def _edge_kernel(src_hbm, dst_hbm, g_hbm, zeros_hbm, out_hbm,
                     src_v, dst_v, rows_v, acc_sh, sem0, sem1):
        cid = lax.axis_index("c")
        sid = lax.axis_index("s")
        wid = cid * NS + sid
        nslice = pl.ds(sid * NODES_PER_TILE, NODES_PER_TILE)
        pltpu.sync_copy(zeros_hbm.at[nslice], acc_sh.at[nslice])
        pltpu.sync_copy(src_hbm.at[wid], src_v)
        pltpu.sync_copy(dst_hbm.at[wid], dst_v)
        plsc.subcore_barrier()

        pltpu.make_async_copy(g_hbm.at[src_v.at[0]], rows_v.at[0], sem0).start()

        def body(i, carry):
            c0 = i * 2
            pltpu.make_async_copy(
                g_hbm.at[src_v.at[c0 + 1]], rows_v.at[1], sem1).start()
            pltpu.make_async_copy(
                g_hbm.at[src_v.at[c0]], rows_v.at[0], sem0).wait()
            pltpu.sync_copy(rows_v.at[0], acc_sh.at[dst_v.at[c0]], add=True)
            pltpu.make_async_copy(
                g_hbm.at[src_v.at[c0 + 2]], rows_v.at[0], sem0).start()
            pltpu.make_async_copy(
                g_hbm.at[src_v.at[c0 + 1]], rows_v.at[1], sem1).wait()
            pltpu.sync_copy(rows_v.at[1], acc_sh.at[dst_v.at[c0 + 1]], add=True)
            return carry

        lax.fori_loop(0, (ROWS_PER_TILE - 1) // 2, body, 0)
        last = ROWS_PER_TILE - 1
        pltpu.make_async_copy(g_hbm.at[src_v.at[last]], rows_v.at[0], sem0).wait()
        pltpu.sync_copy(rows_v.at[0], acc_sh.at[dst_v.at[last]], add=True)

        plsc.subcore_barrier()
        pltpu.sync_copy(acc_sh.at[nslice], out_hbm.at[cid, nslice])

    return _edge_kernel


_edge96 = _make_edge_kernel(96)
_edge64 = _make_edge_kernel(64)
_edge32 = _make_edge_kernel(32)


def _dinv_from(deg_ref):
    deg = deg_ref[0, :, 0:1] + deg_ref[1, :, 0:1] + 1.0
    return lax.rsqrt(deg)


def _m_body(x_ref, w_ref, h_ref):
    h_ref[...] = jnp.dot(x_ref[...], w_ref[...], precision=_HP,
                         preferred_element_type=jnp.float32)


_tc_m = pl.pallas_call(
    _m_body, out_shape=jax.ShapeDtypeStruct((N_PAD, 96), jnp.float32))


def _s_body(h_ref, deg_ref, g_ref):
    g_ref[...] = h_ref[...] * _dinv_from(deg_ref)


_tc_s = pl.pallas_call(
    _s_body, out_shape=jax.ShapeDtypeStruct((N_PAD, 96), jnp.float32))


def _make_tc_d(Dt, Dn):
    def body(acc_ref, g_ref, deg_ref, b_ref, w_ref, out_ref):
        dinv = _dinv_from(deg_ref)
        t = (acc_ref[0] + acc_ref[1] + g_ref[...]) * dinv
        t = jnp.maximum(t[:, :Dt] + b_ref[...], 0.0)
        out_ref[...] = jnp.dot(t, w_ref[...], precision=_HP,
                               preferred_element_type=jnp.float32) * dinv

    return pl.pallas_call(
        body, out_shape=jax.ShapeDtypeStruct((N_PAD, Dn), jnp.float32))


_tc_d1 = _make_tc_d(84, 64)
_tc_d2 = _make_tc_d(64, 32)


def _d3_body(acc_ref, g_ref, deg_ref, b_ref, batch_ref, wl_ref, bl_ref, out_ref):
    dinv = _dinv_from(deg_ref)
    h = (acc_ref[0] + acc_ref[1] + g_ref[...]) * dinv
    h = jnp.maximum(h + b_ref[...], 0.0)
    gid = batch_ref[...]
    oh = (gid == lax.broadcasted_iota(jnp.int32, (1, N_GRAPHS), 1))
    oh = oh.astype(jnp.float32)
    sums = lax.dot_general(oh, h, (((0,), (0,)), ((), ())),
                           precision=_HP, preferred_element_type=jnp.float32)
    cnt = jnp.sum(oh, axis=0).reshape(N_GRAPHS, 1)
    pooled = sums / jnp.maximum(cnt, 1.0)
    out_ref[...] = jnp.dot(pooled, wl_ref[...], precision=_HP,
                           preferred_element_type=jnp.float32) + bl_ref[...]


_tc_d3 = pl.pallas_call(
    _d3_body, out_shape=jax.ShapeDtypeStruct((N_GRAPHS, 10), jnp.float32))


@jax.jit
def kernel(x, edge_index, batch, W1, b1, W2, b2, W3, b3, Wl, bl):
    ei = edge_index.astype(jnp.int32)
    fill = N_NODES + jnp.arange(E_PAD - N_EDGES, dtype=jnp.int32) % (
        N_PAD - N_NODES)
    ei = jnp.concatenate(
        [ei, jnp.broadcast_to(fill, (2, E_PAD - N_EDGES))], axis=1)
    src = ei[0].reshape(NW, ROWS_PER_TILE, CHUNK)
    dst = ei[1].reshape(NW, ROWS_PER_TILE, CHUNK)
    batch2 = jnp.pad(batch.astype(jnp.int32), (0, N_PAD - N_NODES),
                     constant_values=-1).reshape(N_PAD, 1)
    xp = jnp.pad(x, ((0, N_PAD - N_NODES), (0, 0)))

    zdeg = jnp.zeros((N_PAD, DEGW), jnp.float32)
    ones = jnp.ones((CHUNK, DEGW), jnp.float32)
    W1p = jnp.pad(W1, ((0, 0), (0, 96 - 84)))
    h1 = _tc_m(xp, W1p)
    deg2 = _deg_kernel(dst, zdeg, ones)
    g1 = _tc_s(h1, deg2)
    acc1 = _edge96(src, dst, g1, jnp.zeros((N_PAD, 96), jnp.float32))
    g2 = _tc_d1(acc1, g1, deg2, b1.reshape(1, 84), W2)
    acc2 = _edge64(src, dst, g2, jnp.zeros((N_PAD, 64), jnp.float32))
    g3 = _tc_d2(acc2, g2, deg2, b2.reshape(1, 64), W3)
    acc3 = _edge32(src, dst, g3, jnp.zeros((N_PAD, 32), jnp.float32))
    return _tc_d3(acc3, g3, deg2, b3.reshape(1, 32), batch2,
                  Wl, bl.reshape(1, 10))


# speedup vs baseline: 1.9069x; 1.1108x over previous
import functools

import jax
import jax.numpy as jnp
from jax import lax
from jax.experimental import pallas as pl
from jax.experimental.pallas import tpu as pltpu
from jax.experimental.pallas import tpu_sc as plsc

N_NODES = 10000
N_PAD = 10240
N_EDGES = 320000
N_GRAPHS = 64
DW = 128
DEGW = 16

NC = 2
NS = 16
NW = NC * NS
CHUNK = 128
ROWS_PER_TILE = 79
E_PAD = NW * ROWS_PER_TILE * CHUNK
NODES_PER_TILE = N_PAD // NS

_HP = lax.Precision.HIGHEST


def _sc_mesh():
    return plsc.VectorSubcoreMesh(core_axis_name="c", subcore_axis_name="s")


@functools.partial(
    pl.kernel,
    mesh=_sc_mesh(),
    out_type=jax.ShapeDtypeStruct((NC, N_PAD, DEGW), jnp.float32),
    scratch_types=[
        pltpu.VMEM((ROWS_PER_TILE, CHUNK), jnp.int32),
        pltpu.VMEM((CHUNK, DEGW), jnp.float32),
        pltpu.VMEM_SHARED((N_PAD, DEGW), jnp.float32),
        pltpu.SemaphoreType.DMA,
    ],
    compiler_params=pltpu.CompilerParams(use_tc_tiling_on_sc=False),
)
def _deg_kernel(dst_hbm, zeros_hbm, ones_hbm, out_hbm, dst_v, ones_v, acc_sh, sem):
    cid = lax.axis_index("c")
    sid = lax.axis_index("s")
    wid = cid * NS + sid
    nslice = pl.ds(sid * NODES_PER_TILE, NODES_PER_TILE)
    pltpu.sync_copy(zeros_hbm.at[nslice], acc_sh.at[nslice])
    pltpu.sync_copy(dst_hbm.at[wid], dst_v)
    pltpu.sync_copy(ones_hbm, ones_v)
    plsc.subcore_barrier()

    WIN = 8

    def body(c, carry):
        pltpu.make_async_copy(
            ones_v, acc_sh.at[dst_v.at[c]], sem).start(add=True)

        @pl.when(c >= WIN)
        def _():
            pltpu.make_async_copy(ones_v, acc_sh.at[dst_v.at[0]], sem).wait()
        return carry

    lax.fori_loop(0, ROWS_PER_TILE, body, 0)

    def drain(c, carry):
        pltpu.make_async_copy(ones_v, acc_sh.at[dst_v.at[0]], sem).wait()
        return carry

    lax.fori_loop(0, WIN, drain, 0)
    plsc.subcore_barrier()
    pltpu.sync_copy(acc_sh.at[nslice], out_hbm.at[cid, nslice])


def _make_edge_kernel(D):
    @functools.partial(
        pl.kernel,
        mesh=_sc_mesh(),
        out_type=jax.ShapeDtypeStruct((NC, N_PAD, D), jnp.bfloat16),
        scratch_types=[
            pltpu.VMEM((ROWS_PER_TILE, CHUNK), jnp.int32),
            pltpu.VMEM((ROWS_PER_TILE, CHUNK), jnp.int32),
            pltpu.VMEM((2, CHUNK, D), jnp.bfloat16),
            pltpu.VMEM_SHARED((N_PAD, D), jnp.bfloat16),
            pltpu.SemaphoreType.DMA,
            pltpu.SemaphoreType.DMA,
        ],
        compiler_params=pltpu.CompilerParams(use_tc_tiling_on_sc=False),
    )
    def _edge_kernel(src_hbm, dst_hbm, g_hbm, zeros_hbm, out_hbm,
                     src_v, dst_v, rows_v, acc_sh, sem0, sem1):
        cid = lax.axis_index("c")
        sid = lax.axis_index("s")
        wid = cid * NS + sid
        nslice = pl.ds(sid * NODES_PER_TILE, NODES_PER_TILE)
        pltpu.sync_copy(zeros_hbm.at[nslice], acc_sh.at[nslice])
        pltpu.sync_copy(src_hbm.at[wid], src_v)
        pltpu.sync_copy(dst_hbm.at[wid], dst_v)
        plsc.subcore_barrier()

        pltpu.make_async_copy(g_hbm.at[src_v.at[0]], rows_v.at[0], sem0).start()

        def body(i, carry):
            c0 = i * 2
            pltpu.make_async_copy(
                g_hbm.at[src_v.at[c0 + 1]], rows_v.at[1], sem1).start()
            pltpu.make_async_copy(
                g_hbm.at[src_v.at[c0]], rows_v.at[0], sem0).wait()
            pltpu.sync_copy(rows_v.at[0], acc_sh.at[dst_v.at[c0]], add=True)
            pltpu.make_async_copy(
                g_hbm.at[src_v.at[c0 + 2]], rows_v.at[0], sem0).start()
            pltpu.make_async_copy(
                g_hbm.at[src_v.at[c0 + 1]], rows_v.at[1], sem1).wait()
            pltpu.sync_copy(rows_v.at[1], acc_sh.at[dst_v.at[c0 + 1]], add=True)
            return carry

        lax.fori_loop(0, (ROWS_PER_TILE - 1) // 2, body, 0)
        last = ROWS_PER_TILE - 1
        pltpu.make_async_copy(g_hbm.at[src_v.at[last]], rows_v.at[0], sem0).wait()
        pltpu.sync_copy(rows_v.at[0], acc_sh.at[dst_v.at[last]], add=True)

        plsc.subcore_barrier()
        pltpu.sync_copy(acc_sh.at[nslice], out_hbm.at[cid, nslice])

    return _edge_kernel


_edge96 = _make_edge_kernel(96)
_edge64 = _make_edge_kernel(64)
_edge32 = _make_edge_kernel(32)


def _dinv_from(deg_ref):
    deg = deg_ref[0, :, 0:1] + deg_ref[1, :, 0:1] + 1.0
    return lax.rsqrt(deg)


def _m_body(x_ref, w_ref, h_ref):
    h_ref[...] = jnp.dot(x_ref[...], w_ref[...], precision=_HP,
                         preferred_element_type=jnp.float32)


_tc_m = pl.pallas_call(
    _m_body, out_shape=jax.ShapeDtypeStruct((N_PAD, 96), jnp.float32))


def _s_body(h_ref, deg_ref, g_ref):
    g_ref[...] = (h_ref[...] * _dinv_from(deg_ref)).astype(jnp.bfloat16)


_tc_s = pl.pallas_call(
    _s_body, out_shape=jax.ShapeDtypeStruct((N_PAD, 96), jnp.bfloat16))


def _make_tc_d(Dt, Dn):
    def body(acc_ref, g_ref, deg_ref, b_ref, w_ref, out_ref):
        dinv = _dinv_from(deg_ref)
        t = (acc_ref[0].astype(jnp.float32) + acc_ref[1].astype(jnp.float32)
             + g_ref[...].astype(jnp.float32)) * dinv
        t = jnp.maximum(t[:, :Dt] + b_ref[...], 0.0)
        out_ref[...] = (jnp.dot(t, w_ref[...], precision=_HP,
                                preferred_element_type=jnp.float32)
                        * dinv).astype(jnp.bfloat16)

    return pl.pallas_call(
        body, out_shape=jax.ShapeDtypeStruct((N_PAD, Dn), jnp.bfloat16))


_tc_d1 = _make_tc_d(84, 64)
_tc_d2 = _make_tc_d(64, 32)


def _d3_body(acc_ref, g_ref, deg_ref, b_ref, batch_ref, wl_ref, bl_ref, out_ref):
    dinv = _dinv_from(deg_ref)
    h = (acc_ref[0].astype(jnp.float32) + acc_ref[1].astype(jnp.float32)
         + g_ref[...].astype(jnp.float32)) * dinv
    h = jnp.maximum(h + b_ref[...], 0.0)
    gid = batch_ref[...]
    oh = (gid == lax.broadcasted_iota(jnp.int32, (1, N_GRAPHS), 1))
    oh = oh.astype(jnp.float32)
    sums = lax.dot_general(oh, h, (((0,), (0,)), ((), ())),
                           precision=_HP, preferred_element_type=jnp.float32)
    cnt = jnp.sum(oh, axis=0).reshape(N_GRAPHS, 1)
    pooled = sums / jnp.maximum(cnt, 1.0)
    out_ref[...] = jnp.dot(pooled, wl_ref[...], precision=_HP,
                           preferred_element_type=jnp.float32) + bl_ref[...]


_tc_d3 = pl.pallas_call(
    _d3_body, out_shape=jax.ShapeDtypeStruct((N_GRAPHS, 10), jnp.float32))


@jax.jit
def kernel(x, edge_index, batch, W1, b1, W2, b2, W3, b3, Wl, bl):
    ei = edge_index.astype(jnp.int32)
    fill = N_NODES + jnp.arange(E_PAD - N_EDGES, dtype=jnp.int32) % (
        N_PAD - N_NODES)
    ei = jnp.concatenate(
        [ei, jnp.broadcast_to(fill, (2, E_PAD - N_EDGES))], axis=1)
    src = ei[0].reshape(NW, ROWS_PER_TILE, CHUNK)
    dst = ei[1].reshape(NW, ROWS_PER_TILE, CHUNK)
    batch2 = jnp.pad(batch.astype(jnp.int32), (0, N_PAD - N_NODES),
                     constant_values=-1).reshape(N_PAD, 1)
    xp = jnp.pad(x, ((0, N_PAD - N_NODES), (0, 0)))

    zdeg = jnp.zeros((N_PAD, DEGW), jnp.float32)
    ones = jnp.ones((CHUNK, DEGW), jnp.float32)
    W1p = jnp.pad(W1, ((0, 0), (0, 96 - 84)))
    h1 = _tc_m(xp, W1p)
    deg2 = _deg_kernel(dst, zdeg, ones)
    g1 = _tc_s(h1, deg2)
    acc1 = _edge96(src, dst, g1, jnp.zeros((N_PAD, 96), jnp.bfloat16))
    g2 = _tc_d1(acc1, g1, deg2, b1.reshape(1, 84), W2)
    acc2 = _edge64(src, dst, g2, jnp.zeros((N_PAD, 64), jnp.bfloat16))
    g3 = _tc_d2(acc2, g2, deg2, b2.reshape(1, 64), W3)
    acc3 = _edge32(src, dst, g3, jnp.zeros((N_PAD, 32), jnp.bfloat16))
    return _tc_d3(acc3, g3, deg2, b3.reshape(1, 32), batch2,
                  Wl, bl.reshape(1, 10))


# speedup vs baseline: 2.1459x; 1.1253x over previous
import functools

import jax
import jax.numpy as jnp
from jax import lax
from jax.experimental import pallas as pl
from jax.experimental.pallas import tpu as pltpu
from jax.experimental.pallas import tpu_sc as plsc

N_NODES = 10000
N_PAD = 10240
N_EDGES = 320000
N_GRAPHS = 64
DW = 128
DEGW = 16

NC = 2
NS = 16
NW = NC * NS
CHUNK = 128
ROWS_PER_TILE = 80
E_PAD = NW * ROWS_PER_TILE * CHUNK
NODES_PER_TILE = N_PAD // NS

_HP = lax.Precision.HIGHEST


def _sc_mesh():
    return plsc.VectorSubcoreMesh(core_axis_name="c", subcore_axis_name="s")


@functools.partial(
    pl.kernel,
    mesh=_sc_mesh(),
    out_type=jax.ShapeDtypeStruct((NC, N_PAD, DEGW), jnp.float32),
    scratch_types=[
        pltpu.VMEM((ROWS_PER_TILE, CHUNK), jnp.int32),
        pltpu.VMEM((CHUNK, DEGW), jnp.float32),
        pltpu.VMEM_SHARED((N_PAD, DEGW), jnp.float32),
        pltpu.SemaphoreType.DMA,
    ],
    compiler_params=pltpu.CompilerParams(use_tc_tiling_on_sc=False),
)
def _deg_kernel(dst_hbm, zeros_hbm, ones_hbm, out_hbm, dst_v, ones_v, acc_sh, sem):
    cid = lax.axis_index("c")
    sid = lax.axis_index("s")
    wid = cid * NS + sid
    nslice = pl.ds(sid * NODES_PER_TILE, NODES_PER_TILE)
    pltpu.sync_copy(zeros_hbm, acc_sh.at[nslice])
    pltpu.sync_copy(dst_hbm.at[wid], dst_v)
    pltpu.sync_copy(ones_hbm, ones_v)
    plsc.subcore_barrier()

    WIN = 8

    def body(c, carry):
        pltpu.make_async_copy(
            ones_v, acc_sh.at[dst_v.at[c]], sem).start(add=True)

        @pl.when(c >= WIN)
        def _():
            pltpu.make_async_copy(ones_v, acc_sh.at[dst_v.at[0]], sem).wait()
        return carry

    lax.fori_loop(0, ROWS_PER_TILE, body, 0)

    def drain(c, carry):
        pltpu.make_async_copy(ones_v, acc_sh.at[dst_v.at[0]], sem).wait()
        return carry

    lax.fori_loop(0, WIN, drain, 0)
    plsc.subcore_barrier()
    pltpu.sync_copy(acc_sh.at[nslice], out_hbm.at[cid, nslice])


def _make_edge_kernel(D):
    @functools.partial(
        pl.kernel,
        mesh=_sc_mesh(),
        out_type=jax.ShapeDtypeStruct((NC, N_PAD, D), jnp.bfloat16),
        scratch_types=[
            pltpu.VMEM((ROWS_PER_TILE, CHUNK), jnp.int32),
            pltpu.VMEM((ROWS_PER_TILE, CHUNK), jnp.int32),
            pltpu.VMEM((4, CHUNK, D), jnp.bfloat16),
            pltpu.VMEM_SHARED((N_PAD, D), jnp.bfloat16),
            pltpu.SemaphoreType.DMA,
            pltpu.SemaphoreType.DMA,
            pltpu.SemaphoreType.DMA,
            pltpu.SemaphoreType.DMA,
            pltpu.SemaphoreType.DMA,
        ],
        compiler_params=pltpu.CompilerParams(use_tc_tiling_on_sc=False),
    )
    def _edge_kernel(src_hbm, dst_hbm, g_hbm, zeros_hbm, out_hbm,
                     src_v, dst_v, rows_v, acc_sh, g0, g1, g2, g3, ss):
        semg = (g0, g1, g2, g3)
        cid = lax.axis_index("c")
        sid = lax.axis_index("s")
        wid = cid * NS + sid
        nslice = pl.ds(sid * NODES_PER_TILE, NODES_PER_TILE)
        pltpu.sync_copy(zeros_hbm, acc_sh.at[nslice])
        pltpu.sync_copy(src_hbm.at[wid], src_v)
        pltpu.sync_copy(dst_hbm.at[wid], dst_v)
        plsc.subcore_barrier()

        for k in range(4):
            pltpu.make_async_copy(
                g_hbm.at[src_v.at[k]], rows_v.at[k], semg[k]).start()

        def body(i, carry):
            c0 = i * 4
            for k in range(4):
                pltpu.make_async_copy(
                    g_hbm.at[src_v.at[c0 + k]], rows_v.at[k], semg[k]).wait()
                pltpu.make_async_copy(
                    rows_v.at[k], acc_sh.at[dst_v.at[c0 + k]], ss).start(add=True)
            for k in range(4):
                pltpu.make_async_copy(
                    rows_v.at[k], acc_sh.at[dst_v.at[0]], ss).wait()

                @pl.when(c0 + 4 + k < ROWS_PER_TILE)
                def _():
                    pltpu.make_async_copy(
                        g_hbm.at[src_v.at[c0 + 4 + k]], rows_v.at[k],
                        semg[k]).start()
            return carry

        lax.fori_loop(0, ROWS_PER_TILE // 4, body, 0)
        plsc.subcore_barrier()
        pltpu.sync_copy(acc_sh.at[nslice], out_hbm.at[cid, nslice])

    return _edge_kernel


_edge96 = _make_edge_kernel(96)
_edge64 = _make_edge_kernel(64)
_edge32 = _make_edge_kernel(32)


def _dinv_from(deg_ref):
    deg = deg_ref[0, :, 0:1] + deg_ref[1, :, 0:1] + 1.0
    return lax.rsqrt(deg)


def _b_body(x_ref, w_ref, deg_ref, g_ref):
    dinv = _dinv_from(deg_ref)
    h = jnp.dot(x_ref[...], w_ref[...], precision=_HP,
                preferred_element_type=jnp.float32)
    g_ref[0:N_NODES, :] = (h * dinv[0:N_NODES]).astype(jnp.bfloat16)
    g_ref[N_NODES:N_PAD, :] = jnp.zeros(
        (N_PAD - N_NODES, 96), jnp.bfloat16)


_tc_b = pl.pallas_call(
    _b_body, out_shape=jax.ShapeDtypeStruct((N_PAD, 96), jnp.bfloat16))


def _make_tc_d(Dt, Dn):
    def body(acc_ref, g_ref, deg_ref, b_ref, w_ref, out_ref):
        dinv = _dinv_from(deg_ref)
        t = (acc_ref[0].astype(jnp.float32) + acc_ref[1].astype(jnp.float32)
             + g_ref[...].astype(jnp.float32)) * dinv
        t = jnp.maximum(t[:, :Dt] + b_ref[...], 0.0)
        out_ref[...] = (jnp.dot(t, w_ref[...], precision=_HP,
                                preferred_element_type=jnp.float32)
                        * dinv).astype(jnp.bfloat16)

    return pl.pallas_call(
        body, out_shape=jax.ShapeDtypeStruct((N_PAD, Dn), jnp.bfloat16))


_tc_d1 = _make_tc_d(84, 64)
_tc_d2 = _make_tc_d(64, 32)


def _d3_body(acc_ref, g_ref, deg_ref, b_ref, batch_ref, wl_ref, bl_ref, out_ref):
    dinv = _dinv_from(deg_ref)
    h = (acc_ref[0].astype(jnp.float32) + acc_ref[1].astype(jnp.float32)
         + g_ref[...].astype(jnp.float32)) * dinv
    h = jnp.maximum(h + b_ref[...], 0.0)
    gid = batch_ref[...]
    oh = (gid == lax.broadcasted_iota(jnp.int32, (1, N_GRAPHS), 1))
    oh = oh.astype(jnp.float32)
    sums = lax.dot_general(oh, h, (((0,), (0,)), ((), ())),
                           precision=_HP, preferred_element_type=jnp.float32)
    cnt = jnp.sum(oh, axis=0).reshape(N_GRAPHS, 1)
    pooled = sums / jnp.maximum(cnt, 1.0)
    out_ref[...] = jnp.dot(pooled, wl_ref[...], precision=_HP,
                           preferred_element_type=jnp.float32) + bl_ref[...]


_tc_d3 = pl.pallas_call(
    _d3_body, out_shape=jax.ShapeDtypeStruct((N_GRAPHS, 10), jnp.float32))


@jax.jit
def kernel(x, edge_index, batch, W1, b1, W2, b2, W3, b3, Wl, bl):
    ei = edge_index.astype(jnp.int32)
    fill = N_NODES + jnp.arange(E_PAD - N_EDGES, dtype=jnp.int32) % (
        N_PAD - N_NODES)
    ei = jnp.concatenate(
        [ei, jnp.broadcast_to(fill, (2, E_PAD - N_EDGES))], axis=1)
    src = ei[0].reshape(NW, ROWS_PER_TILE, CHUNK)
    dst = ei[1].reshape(NW, ROWS_PER_TILE, CHUNK)
    batch2 = jnp.pad(batch.astype(jnp.int32), (0, N_PAD - N_NODES),
                     constant_values=-1).reshape(N_PAD, 1)

    zdeg = jnp.zeros((NODES_PER_TILE, DEGW), jnp.float32)
    ones = jnp.ones((CHUNK, DEGW), jnp.float32)
    W1p = jnp.pad(W1, ((0, 0), (0, 96 - 84)))
    deg2 = _deg_kernel(dst, zdeg, ones)
    g1 = _tc_b(x, W1p, deg2)
    acc1 = _edge96(src, dst, g1, jnp.zeros((NODES_PER_TILE, 96), jnp.bfloat16))
    g2 = _tc_d1(acc1, g1, deg2, b1.reshape(1, 84), W2)
    acc2 = _edge64(src, dst, g2, jnp.zeros((NODES_PER_TILE, 64), jnp.bfloat16))
    g3 = _tc_d2(acc2, g2, deg2, b2.reshape(1, 64), W3)
    acc3 = _edge32(src, dst, g3, jnp.zeros((NODES_PER_TILE, 32), jnp.bfloat16))
    return _tc_d3(acc3, g3, deg2, b3.reshape(1, 32), batch2,
                  Wl, bl.reshape(1, 10))
